# baseline (speedup 1.0000x reference)
import functools

import jax
import jax.numpy as jnp
from jax import lax
from jax.experimental import pallas as pl
from jax.experimental.pallas import tpu as pltpu
from jax.experimental.pallas import tpu_sc as plsc

B, L = 4096, 200
N = B * L
HD = 192
NC, NS = 2, 16
NW = NC * NS
C = 64
BT = 1024
NSLAB = 4
SL = N // NSLAB

_BF = jnp.bfloat16

_WIDTHS = (64, 64, 64, 120, 120, 104)
_DSTBUF = (0, 0, 1, 2, 3, 4)
_DSTCOL = (0, 64, 0, 0, 0, 0)


def _sc_gather_body(nt, idx_hbm, t_test, t_q, t_tag, t_tg, t_ser,
                    g1, g2, g3, g4, g5,
                    idx_v, r_test, r_q, r_tag, r_tg1, r_tg2, r_ser,
                    isem, gsem, osem):
    tok = nt // NW
    nch = tok // C
    wid = lax.axis_index("s") * NC + lax.axis_index("c")
    base = wid * tok

    tables = (t_test, t_q, t_tag, t_tg, t_tg, t_ser)
    rows = (r_test, r_q, r_tag, r_tg1, r_tg2, r_ser)
    outs = (g1, g2, g3, g4, g5)

    def in_copy(k, g, p):
        off = base + g * C
        return pltpu.make_async_copy(
            idx_hbm.at[pl.ds(k * nt + off, C)], idx_v.at[p, k], isem.at[p])

    def gather_copy(k, p):
        return pltpu.make_async_copy(
            tables[k].at[idx_v.at[p, k]], rows[k].at[p], gsem.at[p])

    def out_copy(k, g, p):
        off = base + g * C
        dst = outs[_DSTBUF[k]].at[pl.ds(off, C),
                                  pl.ds(_DSTCOL[k], _WIDTHS[k])]
        return pltpu.make_async_copy(rows[k].at[p], dst, osem.at[p])

    for k in range(6):
        in_copy(k, 0, 0).start()

    def half(i, p):
        g = 2 * i + p
        q = 1 - p
        if p == 0:
            for k in range(6):
                in_copy(k, g + 1, q).start()
        else:
            @pl.when(i < nch // 2 - 1)
            def _():
                for k in range(6):
                    in_copy(k, g + 1, q).start()
        @pl.when(i >= 1)
        def _():
            for k in range(6):
                out_copy(k, g - 2, p).wait()
        for k in range(6):
            in_copy(k, g, p).wait()
        for k in range(6):
            gather_copy(k, p).start()
        for k in range(6):
            gather_copy(k, p).wait()
        for k in range(6):
            out_copy(k, g, p).start()

    def body(i, carry):
        half(i, 0)
        half(i, 1)
        return carry

    lax.fori_loop(0, nch // 2, body, 0)
    for k in range(6):
        out_copy(k, nch - 2, 0).wait()
    for k in range(6):
        out_copy(k, nch - 1, 1).wait()


def _make_sc_gather(nt):
  return functools.partial(
    pl.kernel,
    out_type=[jax.ShapeDtypeStruct((nt, 128), jnp.float32)] * 5,
    mesh=plsc.VectorSubcoreMesh(core_axis_name="c", subcore_axis_name="s"),
    compiler_params=pltpu.CompilerParams(use_tc_tiling_on_sc=False),
    scratch_types=[
        pltpu.VMEM((2, 6, C), jnp.int32),
        pltpu.VMEM((2, C, 64), jnp.float32),
        pltpu.VMEM((2, C, 64), jnp.float32),
        pltpu.VMEM((2, C, 64), jnp.float32),
        pltpu.VMEM((2, C, 120), jnp.float32),
        pltpu.VMEM((2, C, 120), jnp.float32),
        pltpu.VMEM((2, C, 104), jnp.float32),
        pltpu.SemaphoreType.DMA((2,)),
        pltpu.SemaphoreType.DMA((2,)),
        pltpu.SemaphoreType.DMA((2,)),
    ],
)(functools.partial(_sc_gather_body, nt))


_sc_gather = _make_sc_gather(SL)


def _tc_body(ii_ref, s_ref, g1_ref, g2_ref, g3_ref, g4_ref, g5_ref,
             ei_ref, wtq_ref, wg_ref, w1_ref, w2_ref, ws_ref,
             wi_ref, wsc_ref, o_ref):
    f32 = jnp.float32
    acc = jnp.dot(g1_ref[...].astype(_BF), wtq_ref[...],
                  preferred_element_type=f32)
    acc += jnp.dot(g2_ref[:, :64].astype(_BF), wg_ref[...],
                   preferred_element_type=f32)
    acc += jnp.dot(g3_ref[:, :120].astype(_BF), w1_ref[...],
                   preferred_element_type=f32)
    acc += jnp.dot(g4_ref[:, :120].astype(_BF), w2_ref[...],
                   preferred_element_type=f32)
    acc += jnp.dot(g5_ref[:, :104].astype(_BF), ws_ref[...],
                   preferred_element_type=f32)
    ii = ii_ref[0, 0, :]
    oh = (ii[:, None] == lax.broadcasted_iota(jnp.int32, (BT, 8), 1)).astype(_BF)
    e_int = jnp.dot(oh, ei_ref[...], preferred_element_type=f32)
    acc += jnp.dot(e_int.astype(_BF), wi_ref[...], preferred_element_type=f32)
    acc += lax.dot_general(s_ref[...], wsc_ref[...], (((0,), (0,)), ((), ())),
                           preferred_element_type=f32)
    o_ref[...] = acc


def _tc_project(ii3, s_feat, g1, g2, g3, g4, g5, ei_pad,
                w_tq, w_tag, w_tg1a, w_tg1b, w_ser, w_int, w_scal):
    nt = g1.shape[0]
    rep = lambda shp: pl.BlockSpec(shp, lambda i: (0, 0))
    return pl.pallas_call(
        _tc_body,
        grid=(nt // BT,),
        in_specs=[
            pl.BlockSpec((1, 1, BT), lambda i: (i, 0, 0)),
            pl.BlockSpec((8, BT), lambda i: (0, i)),
            pl.BlockSpec((BT, 128), lambda i: (i, 0)),
            pl.BlockSpec((BT, 128), lambda i: (i, 0)),
            pl.BlockSpec((BT, 128), lambda i: (i, 0)),
            pl.BlockSpec((BT, 128), lambda i: (i, 0)),
            pl.BlockSpec((BT, 128), lambda i: (i, 0)),
            rep((8, 64)),
            rep((128, HD)),
            rep((64, HD)),
            rep((120, HD)),
            rep((120, HD)),
            rep((104, HD)),
            rep((64, HD)),
            rep((8, HD)),
        ],
        out_specs=pl.BlockSpec((BT, HD), lambda i: (i, 0)),
        out_shape=jax.ShapeDtypeStruct((nt, HD), jnp.float32),
    )(ii3, s_feat, g1, g2, g3, g4, g5, ei_pad,
      w_tq, w_tag, w_tg1a, w_tg1b, w_ser, w_int, w_scal)


def kernel(test, question, tag, correct, mask, interaction, duration,
           test_group_one, test_group_two, serial, solved_count,
           correct_before, wrong_before, same_tag_solved_count,
           same_tag_correct_before, same_tag_wrong_before,
           emb_interaction, emb_test, emb_question, emb_tag,
           emb_tg1, emb_serial, W, b):
    f32 = jnp.float32
    ids = (test.reshape(N), question.reshape(N), tag.reshape(N),
           test_group_one.reshape(N), test_group_two.reshape(N),
           serial.reshape(N))

    t_tg = jnp.pad(emb_tg1, ((0, 0), (0, 5)))
    t_ser = jnp.pad(emb_serial, ((0, 0), (0, 4)))
    gs = []
    for s in range(NSLAB):
        idx_s = jnp.concatenate([a[s * SL:(s + 1) * SL] for a in ids], axis=0)
        gs.append(_sc_gather(idx_s, emb_test, emb_question, emb_tag,
                             t_tg, t_ser))

    w_int = W[0:64].astype(_BF)
    w_tq = W[64:192].astype(_BF)
    w_tag = W[192:256].astype(_BF)
    w_tg1a = jnp.pad(W[257:372], ((0, 5), (0, 0))).astype(_BF)
    w_tg1b = jnp.pad(W[372:487], ((0, 5), (0, 0))).astype(_BF)
    w_ser = jnp.pad(W[487:587], ((0, 4), (0, 0))).astype(_BF)
    w_scal = jnp.concatenate([W[256:257], W[587:593], b[None, :]], axis=0)
    s_feat = jnp.stack([
        duration.reshape(N),
        solved_count.reshape(N).astype(f32),
        correct_before.reshape(N).astype(f32),
        wrong_before.reshape(N).astype(f32),
        same_tag_solved_count.reshape(N).astype(f32),
        same_tag_correct_before.reshape(N).astype(f32),
        same_tag_wrong_before.reshape(N).astype(f32),
        jnp.ones((N,), f32),
    ], axis=0)
    ei_pad = jnp.pad(emb_interaction, ((0, 5), (0, 0))).astype(_BF)
    ii3 = interaction.reshape(N // BT, 1, BT)

    xs = []
    for s in range(NSLAB):
        nb = SL // BT
        ii3_s = ii3[s * nb:(s + 1) * nb]
        s_feat_s = s_feat[:, s * SL:(s + 1) * SL]
        g1, g2, g3, g4, g5 = gs[s]
        xs.append(_tc_project(ii3_s, s_feat_s, g1, g2, g3, g4, g5, ei_pad,
                              w_tq, w_tag, w_tg1a, w_tg1b, w_ser,
                              w_int, w_scal))
    X = jnp.concatenate(xs, axis=0)
    return X.reshape(B, L, HD), interaction.shape[0]


# speedup vs baseline: 1.0316x; 1.0316x over previous
import functools

import jax
import jax.numpy as jnp
from jax import lax
from jax.experimental import pallas as pl
from jax.experimental.pallas import tpu as pltpu
from jax.experimental.pallas import tpu_sc as plsc

B, L = 4096, 200
N = B * L
HD = 192
NC, NS = 2, 16
NW = NC * NS
TOK = N // NW
C = 64
NCH = TOK // C
BT = 1024

_BF = jnp.bfloat16

_GATHW = (128, 128, 128, 120, 120, 104)
_WIDTHS = (64, 64, 64, 120, 120, 104)
_DSTBUF = (0, 0, 1, 2, 3, 4)
_DSTCOL = (0, 64, 0, 0, 0, 0)


def _sc_gather_body(idx_hbm, t_test, t_q, t_tag, t_tg, t_ser,
                    g1, g2, g3, g4, g5,
                    idx_v, r_test, r_q, r_tag, r_tg1, r_tg2, r_ser,
                    isem, gsem, osem):
    wid = lax.axis_index("s") * NC + lax.axis_index("c")
    base = wid * TOK

    tables = (t_test, t_q, t_tag, t_tg, t_tg, t_ser)
    rows = (r_test, r_q, r_tag, r_tg1, r_tg2, r_ser)
    outs = (g1, g2, g3, g4, g5)

    def in_copy(k, g, p):
        off = base + g * C
        return pltpu.make_async_copy(
            idx_hbm.at[pl.ds(k * N + off, C)], idx_v.at[p, k], isem.at[p])

    def gather_copy(k, p):
        return pltpu.make_async_copy(
            tables[k].at[idx_v.at[p, k]], rows[k].at[p], gsem.at[p])

    def out_copy(k, g, p):
        off = base + g * C
        src_r = rows[k].at[p]
        if _GATHW[k] != _WIDTHS[k]:
            src_r = rows[k].at[p, :, pl.ds(0, _WIDTHS[k])]
        dst = outs[_DSTBUF[k]].at[pl.ds(off, C),
                                  pl.ds(_DSTCOL[k], _WIDTHS[k])]
        return pltpu.make_async_copy(src_r, dst, osem.at[p])

    for k in range(6):
        in_copy(k, 0, 0).start()

    def half(i, p):
        g = 2 * i + p
        q = 1 - p
        if p == 0:
            for k in range(6):
                in_copy(k, g + 1, q).start()
        else:
            @pl.when(i < NCH // 2 - 1)
            def _():
                for k in range(6):
                    in_copy(k, g + 1, q).start()
        @pl.when(i >= 1)
        def _():
            for k in range(6):
                out_copy(k, g - 2, p).wait()
        for k in range(6):
            in_copy(k, g, p).wait()
        for k in range(6):
            gather_copy(k, p).start()
        for k in range(6):
            gather_copy(k, p).wait()
        for k in range(6):
            out_copy(k, g, p).start()

    def body(i, carry):
        half(i, 0)
        half(i, 1)
        return carry

    lax.fori_loop(0, NCH // 2, body, 0)
    for k in range(6):
        out_copy(k, NCH - 2, 0).wait()
    for k in range(6):
        out_copy(k, NCH - 1, 1).wait()


_sc_gather = functools.partial(
    pl.kernel,
    out_type=[jax.ShapeDtypeStruct((N, 128), jnp.float32)] * 5,
    mesh=plsc.VectorSubcoreMesh(core_axis_name="c", subcore_axis_name="s"),
    compiler_params=pltpu.CompilerParams(use_tc_tiling_on_sc=False),
    scratch_types=[
        pltpu.VMEM((2, 6, C), jnp.int32),
        pltpu.VMEM((2, C, 128), jnp.float32),
        pltpu.VMEM((2, C, 128), jnp.float32),
        pltpu.VMEM((2, C, 128), jnp.float32),
        pltpu.VMEM((2, C, 120), jnp.float32),
        pltpu.VMEM((2, C, 120), jnp.float32),
        pltpu.VMEM((2, C, 104), jnp.float32),
        pltpu.SemaphoreType.DMA((2,)),
        pltpu.SemaphoreType.DMA((2,)),
        pltpu.SemaphoreType.DMA((2,)),
    ],
)(_sc_gather_body)


def _tc_body(ii_ref, s_ref, g1_ref, g2_ref, g3_ref, g4_ref, g5_ref,
             ei_ref, wtq_ref, wg_ref, w1_ref, w2_ref, ws_ref,
             wi_ref, wsc_ref, o_ref):
    f32 = jnp.float32
    acc = jnp.dot(g1_ref[...].astype(_BF), wtq_ref[...],
                  preferred_element_type=f32)
    acc += jnp.dot(g2_ref[:, :64].astype(_BF), wg_ref[...],
                   preferred_element_type=f32)
    acc += jnp.dot(g3_ref[:, :120].astype(_BF), w1_ref[...],
                   preferred_element_type=f32)
    acc += jnp.dot(g4_ref[:, :120].astype(_BF), w2_ref[...],
                   preferred_element_type=f32)
    acc += jnp.dot(g5_ref[:, :104].astype(_BF), ws_ref[...],
                   preferred_element_type=f32)
    ii = ii_ref[0, 0, :]
    oh = (ii[:, None] == lax.broadcasted_iota(jnp.int32, (BT, 8), 1)).astype(_BF)
    e_int = jnp.dot(oh, ei_ref[...], preferred_element_type=f32)
    acc += jnp.dot(e_int.astype(_BF), wi_ref[...], preferred_element_type=f32)
    acc += lax.dot_general(s_ref[...], wsc_ref[...], (((0,), (0,)), ((), ())),
                           preferred_element_type=f32)
    o_ref[...] = acc


def _tc_project(ii3, s_feat, g1, g2, g3, g4, g5, ei_pad,
                w_tq, w_tag, w_tg1a, w_tg1b, w_ser, w_int, w_scal):
    rep = lambda shp: pl.BlockSpec(shp, lambda i: (0, 0))
    return pl.pallas_call(
        _tc_body,
        grid=(N // BT,),
        in_specs=[
            pl.BlockSpec((1, 1, BT), lambda i: (i, 0, 0)),
            pl.BlockSpec((8, BT), lambda i: (0, i)),
            pl.BlockSpec((BT, 128), lambda i: (i, 0)),
            pl.BlockSpec((BT, 128), lambda i: (i, 0)),
            pl.BlockSpec((BT, 128), lambda i: (i, 0)),
            pl.BlockSpec((BT, 128), lambda i: (i, 0)),
            pl.BlockSpec((BT, 128), lambda i: (i, 0)),
            rep((8, 64)),
            rep((128, HD)),
            rep((64, HD)),
            rep((120, HD)),
            rep((120, HD)),
            rep((104, HD)),
            rep((64, HD)),
            rep((8, HD)),
        ],
        out_specs=pl.BlockSpec((BT, HD), lambda i: (i, 0)),
        out_shape=jax.ShapeDtypeStruct((N, HD), jnp.float32),
    )(ii3, s_feat, g1, g2, g3, g4, g5, ei_pad,
      w_tq, w_tag, w_tg1a, w_tg1b, w_ser, w_int, w_scal)


def kernel(test, question, tag, correct, mask, interaction, duration,
           test_group_one, test_group_two, serial, solved_count,
           correct_before, wrong_before, same_tag_solved_count,
           same_tag_correct_before, same_tag_wrong_before,
           emb_interaction, emb_test, emb_question, emb_tag,
           emb_tg1, emb_serial, W, b):
    f32 = jnp.float32
    idx = jnp.concatenate([
        test.reshape(N), question.reshape(N), tag.reshape(N),
        test_group_one.reshape(N), test_group_two.reshape(N),
        serial.reshape(N),
    ], axis=0)

    pad128 = lambda t: jnp.pad(t, ((0, 0), (0, 64)))
    t_tg = jnp.pad(emb_tg1, ((0, 0), (0, 5)))
    t_ser = jnp.pad(emb_serial, ((0, 0), (0, 4)))
    g1, g2, g3, g4, g5 = _sc_gather(
        idx, pad128(emb_test), pad128(emb_question), pad128(emb_tag),
        t_tg, t_ser)

    w_int = W[0:64].astype(_BF)
    w_tq = W[64:192].astype(_BF)
    w_tag = W[192:256].astype(_BF)
    w_tg1a = jnp.pad(W[257:372], ((0, 5), (0, 0))).astype(_BF)
    w_tg1b = jnp.pad(W[372:487], ((0, 5), (0, 0))).astype(_BF)
    w_ser = jnp.pad(W[487:587], ((0, 4), (0, 0))).astype(_BF)
    w_scal = jnp.concatenate([W[256:257], W[587:593], b[None, :]], axis=0)
    s_feat = jnp.stack([
        duration.reshape(N),
        solved_count.reshape(N).astype(f32),
        correct_before.reshape(N).astype(f32),
        wrong_before.reshape(N).astype(f32),
        same_tag_solved_count.reshape(N).astype(f32),
        same_tag_correct_before.reshape(N).astype(f32),
        same_tag_wrong_before.reshape(N).astype(f32),
        jnp.ones((N,), f32),
    ], axis=0)
    ei_pad = jnp.pad(emb_interaction, ((0, 5), (0, 0))).astype(_BF)
    ii3 = interaction.reshape(N // BT, 1, BT)

    X = _tc_project(ii3, s_feat, g1, g2, g3, g4, g5, ei_pad,
                    w_tq, w_tag, w_tg1a, w_tg1b, w_ser, w_int, w_scal)
    return X.reshape(B, L, HD), interaction.shape[0]


# speedup vs baseline: 1.1027x; 1.0689x over previous
import functools

import jax
import jax.numpy as jnp
from jax import lax
from jax.experimental import pallas as pl
from jax.experimental.pallas import tpu as pltpu
from jax.experimental.pallas import tpu_sc as plsc

B, L = 4096, 200
N = B * L
HD = 192
NC, NS = 2, 16
NW = NC * NS
TOK = N // NW
C = 64
NCH = TOK // C
BT = 1024

_BF = jnp.bfloat16

_WIDTHS = (64, 64, 64, 120, 120, 104)
_DSTBUF = (0, 0, 1, 2, 3, 4)
_DSTCOL = (0, 64, 0, 0, 0, 0)


def _sc_gather_body(idx_hbm, t_test, t_q, t_tag, t_tg, t_ser,
                    g1, g2, g3, g4, g5,
                    idx_v, r_test, r_q, r_tag, r_tg1, r_tg2, r_ser,
                    isem, gsem, osem):
    wid = lax.axis_index("s") * NC + lax.axis_index("c")
    base = wid * TOK

    tables = (t_test, t_q, t_tag, t_tg, t_tg, t_ser)
    rows = (r_test, r_q, r_tag, r_tg1, r_tg2, r_ser)
    outs = (g1, g2, g3, g4, g5)

    def in_copy(k, g, p):
        off = base + g * C
        return pltpu.make_async_copy(
            idx_hbm.at[pl.ds(k * N + off, C)], idx_v.at[p, k], isem.at[p])

    def gather_copy(k, p):
        return pltpu.make_async_copy(
            tables[k].at[idx_v.at[p, k]], rows[k].at[p], gsem.at[p])

    def out_copy(k, g, p):
        off = base + g * C
        dst = outs[_DSTBUF[k]].at[pl.ds(off, C),
                                  pl.ds(_DSTCOL[k], _WIDTHS[k])]
        return pltpu.make_async_copy(rows[k].at[p], dst, osem.at[p])

    for k in range(6):
        in_copy(k, 0, 0).start()

    def half(i, p):
        g = 2 * i + p
        q = 1 - p
        if p == 0:
            for k in range(6):
                in_copy(k, g + 1, q).start()
        else:
            @pl.when(i < NCH // 2 - 1)
            def _():
                for k in range(6):
                    in_copy(k, g + 1, q).start()
        @pl.when(i >= 1)
        def _():
            for k in range(6):
                out_copy(k, g - 2, p).wait()
        for k in range(6):
            in_copy(k, g, p).wait()
        for k in range(6):
            gather_copy(k, p).start()
        for k in range(6):
            gather_copy(k, p).wait()
        for k in range(6):
            out_copy(k, g, p).start()

    def body(i, carry):
        half(i, 0)
        half(i, 1)
        return carry

    lax.fori_loop(0, NCH // 2, body, 0)
    for k in range(6):
        out_copy(k, NCH - 2, 0).wait()
    for k in range(6):
        out_copy(k, NCH - 1, 1).wait()


_sc_gather = functools.partial(
    pl.kernel,
    out_type=[jax.ShapeDtypeStruct((N, 128), jnp.float32)] * 5,
    mesh=plsc.VectorSubcoreMesh(core_axis_name="c", subcore_axis_name="s"),
    compiler_params=pltpu.CompilerParams(use_tc_tiling_on_sc=False),
    scratch_types=[
        pltpu.VMEM((2, 6, C), jnp.int32),
        pltpu.VMEM((2, C, 64), jnp.float32),
        pltpu.VMEM((2, C, 64), jnp.float32),
        pltpu.VMEM((2, C, 64), jnp.float32),
        pltpu.VMEM((2, C, 120), jnp.float32),
        pltpu.VMEM((2, C, 120), jnp.float32),
        pltpu.VMEM((2, C, 104), jnp.float32),
        pltpu.SemaphoreType.DMA((2,)),
        pltpu.SemaphoreType.DMA((2,)),
        pltpu.SemaphoreType.DMA((2,)),
    ],
)(_sc_gather_body)


def _tc_body(ii_ref, s_ref, g1_ref, g2_ref, g3_ref, g4_ref, g5_ref,
             ei_ref, wtq_ref, wg_ref, w1_ref, w2_ref, ws_ref,
             wi_ref, wsc_ref, o_ref):
    f32 = jnp.float32
    acc = jnp.dot(g1_ref[...].astype(_BF), wtq_ref[...],
                  preferred_element_type=f32)
    acc += jnp.dot(g2_ref[:, :64].astype(_BF), wg_ref[...],
                   preferred_element_type=f32)
    acc += jnp.dot(g3_ref[:, :120].astype(_BF), w1_ref[...],
                   preferred_element_type=f32)
    acc += jnp.dot(g4_ref[:, :120].astype(_BF), w2_ref[...],
                   preferred_element_type=f32)
    acc += jnp.dot(g5_ref[:, :104].astype(_BF), ws_ref[...],
                   preferred_element_type=f32)
    ii = ii_ref[0, 0, :]
    oh = (ii[:, None] == lax.broadcasted_iota(jnp.int32, (BT, 8), 1)).astype(_BF)
    e_int = jnp.dot(oh, ei_ref[...], preferred_element_type=f32)
    acc += jnp.dot(e_int.astype(_BF), wi_ref[...], preferred_element_type=f32)
    acc += lax.dot_general(s_ref[...], wsc_ref[...], (((0,), (0,)), ((), ())),
                           preferred_element_type=f32)
    o_ref[...] = acc


def _tc_project(ii3, s_feat, g1, g2, g3, g4, g5, ei_pad,
                w_tq, w_tag, w_tg1a, w_tg1b, w_ser, w_int, w_scal):
    rep = lambda shp: pl.BlockSpec(shp, lambda i: (0, 0))
    return pl.pallas_call(
        _tc_body,
        grid=(N // BT,),
        in_specs=[
            pl.BlockSpec((1, 1, BT), lambda i: (i, 0, 0)),
            pl.BlockSpec((8, BT), lambda i: (0, i)),
            pl.BlockSpec((BT, 128), lambda i: (i, 0)),
            pl.BlockSpec((BT, 128), lambda i: (i, 0)),
            pl.BlockSpec((BT, 128), lambda i: (i, 0)),
            pl.BlockSpec((BT, 128), lambda i: (i, 0)),
            pl.BlockSpec((BT, 128), lambda i: (i, 0)),
            rep((8, 64)),
            rep((128, HD)),
            rep((64, HD)),
            rep((120, HD)),
            rep((120, HD)),
            rep((104, HD)),
            rep((64, HD)),
            rep((8, HD)),
        ],
        out_specs=pl.BlockSpec((BT, HD), lambda i: (i, 0)),
        out_shape=jax.ShapeDtypeStruct((N, HD), jnp.float32),
    )(ii3, s_feat, g1, g2, g3, g4, g5, ei_pad,
      w_tq, w_tag, w_tg1a, w_tg1b, w_ser, w_int, w_scal)


def kernel(test, question, tag, correct, mask, interaction, duration,
           test_group_one, test_group_two, serial, solved_count,
           correct_before, wrong_before, same_tag_solved_count,
           same_tag_correct_before, same_tag_wrong_before,
           emb_interaction, emb_test, emb_question, emb_tag,
           emb_tg1, emb_serial, W, b):
    f32 = jnp.float32
    idx = jnp.concatenate([
        test.reshape(N), question.reshape(N), tag.reshape(N),
        test_group_one.reshape(N), test_group_two.reshape(N),
        serial.reshape(N),
    ], axis=0)

    t_tg = jnp.pad(emb_tg1, ((0, 0), (0, 5)))
    t_ser = jnp.pad(emb_serial, ((0, 0), (0, 4)))
    g1, g2, g3, g4, g5 = _sc_gather(
        idx, emb_test, emb_question, emb_tag, t_tg, t_ser)

    w_int = W[0:64].astype(_BF)
    w_tq = W[64:192].astype(_BF)
    w_tag = W[192:256].astype(_BF)
    w_tg1a = jnp.pad(W[257:372], ((0, 5), (0, 0))).astype(_BF)
    w_tg1b = jnp.pad(W[372:487], ((0, 5), (0, 0))).astype(_BF)
    w_ser = jnp.pad(W[487:587], ((0, 4), (0, 0))).astype(_BF)
    w_scal = jnp.concatenate([W[256:257], W[587:593], b[None, :]], axis=0)
    s_feat = jnp.stack([
        duration.reshape(N),
        solved_count.reshape(N).astype(f32),
        correct_before.reshape(N).astype(f32),
        wrong_before.reshape(N).astype(f32),
        same_tag_solved_count.reshape(N).astype(f32),
        same_tag_correct_before.reshape(N).astype(f32),
        same_tag_wrong_before.reshape(N).astype(f32),
        jnp.ones((N,), f32),
    ], axis=0)
    ei_pad = jnp.pad(emb_interaction, ((0, 5), (0, 0))).astype(_BF)
    ii3 = interaction.reshape(N // BT, 1, BT)

    X = _tc_project(ii3, s_feat, g1, g2, g3, g4, g5, ei_pad,
                    w_tq, w_tag, w_tg1a, w_tg1b, w_ser, w_int, w_scal)
    return X.reshape(B, L, HD), interaction.shape[0]
